# baseline (speedup 1.0000x reference)
import functools

import jax
import jax.numpy as jnp
from jax import lax
from jax.experimental import pallas as pl
from jax.experimental.pallas import tpu as pltpu
from jax.experimental.pallas import tpu_sc as plsc

EMB_DIM = 256
CODEBOOK = 8192
TOKENS = 8192
TB = 1024
CB = 1024
NT = TOKENS // TB
NCB = CODEBOOK // CB
NSTEPS = NT * NCB
ZROWS = TOKENS // NSTEPS
LOSS_SCALE = 1.25 / (TOKENS * EMB_DIM)


def _stage1_body(x_ref, e_ref, idx_ref, loss_ref, enc_ref,
                 minv, mini, acc, zbuf, zsem):
    i = pl.program_id(0)
    j = pl.program_id(1)
    t = i * NCB + j

    @pl.when(t == 0)
    def _():
        acc[0] = jnp.float32(0.0)
        zbuf[...] = jnp.zeros((ZROWS, CODEBOOK), jnp.float32)

    pltpu.make_async_copy(
        zbuf, enc_ref.at[pl.ds(t * ZROWS, ZROWS)], zsem).start()

    @pl.when(t > 0)
    def _():
        pltpu.make_async_copy(
            zbuf, enc_ref.at[pl.ds((t - 1) * ZROWS, ZROWS)], zsem).wait()

    @pl.when(t == NSTEPS - 1)
    def _():
        pltpu.make_async_copy(
            zbuf, enc_ref.at[pl.ds(t * ZROWS, ZROWS)], zsem).wait()

    @pl.when(j == 0)
    def _():
        minv[...] = jnp.full((TB, 128), jnp.inf, jnp.float32)
        mini[...] = jnp.zeros((TB, 128), jnp.int32)

    xb = x_ref[...]
    eb = e_ref[...]
    s = lax.dot_general(xb, eb, (((1,), (0,)), ((), ())),
                        preferred_element_type=jnp.float32)
    a = jnp.sum(xb * xb, axis=1, keepdims=True)
    b = jnp.sum(eb * eb, axis=0)
    m = minv[...]
    ii = mini[...]
    for k in range(CB // 128):
        sk = lax.slice(s, (0, k * 128), (TB, (k + 1) * 128))
        bk = lax.slice(b, (k * 128,), ((k + 1) * 128,))
        dk = (a + bk) - 2.0 * sk
        cond = dk < m
        m = jnp.where(cond, dk, m)
        ii = jnp.where(cond, jnp.int32(j * (CB // 128) + k), ii)
    minv[...] = m
    mini[...] = ii

    @pl.when(j == NCB - 1)
    def _():
        lane = lax.broadcasted_iota(jnp.int32, (TB, 128), 1)
        gmin = jnp.min(m, axis=1)
        cand = jnp.where(m == gmin[:, None], ii * 128 + lane,
                         jnp.int32(0x7FFFFFFF))
        idx_ref[...] = jnp.min(cand, axis=1)
        acc[0] = acc[0] + jnp.sum(gmin)

    @pl.when(t == NSTEPS - 1)
    def _():
        loss_ref[0, 0] = acc[0] * LOSS_SCALE


def _argmin_loss_zeros(x, emb):
    return pl.pallas_call(
        _stage1_body,
        grid=(NT, NCB),
        in_specs=[
            pl.BlockSpec((TB, EMB_DIM), lambda i, j: (i, 0)),
            pl.BlockSpec((EMB_DIM, CB), lambda i, j: (0, j)),
        ],
        out_specs=[
            pl.BlockSpec((TB,), lambda i, j: (i,)),
            pl.BlockSpec(memory_space=pltpu.SMEM),
            pl.BlockSpec(memory_space=pl.ANY),
        ],
        out_shape=[
            jax.ShapeDtypeStruct((TOKENS,), jnp.int32),
            jax.ShapeDtypeStruct((1, 1), jnp.float32),
            jax.ShapeDtypeStruct((TOKENS, CODEBOOK), jnp.float32),
        ],
        scratch_shapes=[
            pltpu.VMEM((TB, 128), jnp.float32),
            pltpu.VMEM((TB, 128), jnp.int32),
            pltpu.SMEM((1,), jnp.float32),
            pltpu.VMEM((ZROWS, CODEBOOK), jnp.float32),
            pltpu.SemaphoreType.DMA,
        ],
        compiler_params=pltpu.CompilerParams(
            dimension_semantics=("arbitrary", "arbitrary")),
        name="stage1",
    )(x, emb)


def _sc_gather(table, idx):
    info = plsc.get_sparse_core_info()
    nc, ns = info.num_cores, info.num_subcores
    nw = nc * ns
    b_per_w = TOKENS // nw
    mesh = plsc.VectorSubcoreMesh(core_axis_name="c", subcore_axis_name="s")

    @functools.partial(
        pl.kernel, mesh=mesh,
        out_type=jax.ShapeDtypeStruct((TOKENS, EMB_DIM), jnp.float32),
        scratch_types=[
            pltpu.VMEM((b_per_w,), jnp.int32),
            pltpu.VMEM((b_per_w, EMB_DIM), jnp.float32),
            pltpu.SemaphoreType.DMA,
        ],
    )
    def gather_k(table_hbm, idx_hbm, out_hbm, idx_v, rows_v, sem):
        wid = lax.axis_index("s") * nc + lax.axis_index("c")
        base = wid * b_per_w
        pltpu.sync_copy(idx_hbm.at[pl.ds(base, b_per_w)], idx_v)
        pltpu.async_copy(table_hbm.at[idx_v], rows_v, sem).wait()
        pltpu.sync_copy(rows_v, out_hbm.at[pl.ds(base, b_per_w)])

    return gather_k(table, idx)


def kernel(inputs, embeddings):
    x = inputs.reshape(-1, EMB_DIM)
    idx, loss11, enc0 = _argmin_loss_zeros(x, embeddings)
    emb_t = jnp.swapaxes(embeddings, 0, 1)
    quantized = _sc_gather(emb_t, idx)
    encodings = enc0.at[jnp.arange(TOKENS), idx].set(1.0)
    quantized_st = quantized.reshape(inputs.shape)
    encoding_indices = idx.reshape(inputs.shape[:-1])
    loss = loss11[0, 0]
    return quantized_st, encodings, encoding_indices, loss


# speedup vs baseline: 3.3019x; 3.3019x over previous
import functools

import jax
import jax.numpy as jnp
from jax import lax
from jax.experimental import pallas as pl
from jax.experimental.pallas import tpu as pltpu
from jax.experimental.pallas import tpu_sc as plsc

EMB_DIM = 256
CODEBOOK = 8192
TOKENS = 8192
TB = 1024
CB = 1024
NT = TOKENS // TB
NCB = CODEBOOK // CB
NSTEPS = NT * NCB
ZROWS = TOKENS // NSTEPS
LOSS_SCALE = 1.25 / (TOKENS * EMB_DIM)


def _stage1_body(x_ref, e_ref, idx_ref, loss_ref, enc_ref, row64_ref,
                 minv, mini, acc, zbuf, zsem):
    i = pl.program_id(0)
    j = pl.program_id(1)
    t = i * NCB + j

    @pl.when(t == 0)
    def _():
        acc[0] = jnp.float32(0.0)
        zbuf[...] = jnp.zeros((ZROWS, CODEBOOK), jnp.float32)

    pltpu.make_async_copy(
        zbuf, enc_ref.at[pl.ds(t * ZROWS, ZROWS)], zsem).start()

    @pl.when(t > 0)
    def _():
        pltpu.make_async_copy(
            zbuf, enc_ref.at[pl.ds((t - 1) * ZROWS, ZROWS)], zsem).wait()

    @pl.when(t == NSTEPS - 1)
    def _():
        pltpu.make_async_copy(
            zbuf, enc_ref.at[pl.ds(t * ZROWS, ZROWS)], zsem).wait()

    @pl.when(j == 0)
    def _():
        minv[...] = jnp.full((TB, 128), jnp.inf, jnp.float32)
        mini[...] = jnp.zeros((TB, 128), jnp.int32)

    xb = x_ref[...]
    eb = e_ref[...]
    s = lax.dot_general(xb, eb, (((1,), (0,)), ((), ())),
                        preferred_element_type=jnp.float32)
    a = jnp.sum(xb * xb, axis=1, keepdims=True)
    b = jnp.sum(eb * eb, axis=0)
    m = minv[...]
    ii = mini[...]
    for k in range(CB // 128):
        sk = lax.slice(s, (0, k * 128), (TB, (k + 1) * 128))
        bk = lax.slice(b, (k * 128,), ((k + 1) * 128,))
        dk = (a + bk) - 2.0 * sk
        cond = dk < m
        m = jnp.where(cond, dk, m)
        ii = jnp.where(cond, jnp.int32(j * (CB // 128) + k), ii)
    minv[...] = m
    mini[...] = ii

    @pl.when(j == NCB - 1)
    def _():
        lane = lax.broadcasted_iota(jnp.int32, (TB, 128), 1)
        gmin = jnp.min(m, axis=1)
        cand = jnp.where(m == gmin[:, None], ii * 128 + lane,
                         jnp.int32(0x7FFFFFFF))
        win = jnp.min(cand, axis=1)
        idx_ref[...] = win
        toks = i * TB + lax.broadcasted_iota(jnp.int32, (TB,), 0)
        row64_ref[...] = toks * 64 + (win >> 7)
        acc[0] = acc[0] + jnp.sum(gmin)

    @pl.when(t == NSTEPS - 1)
    def _():
        loss_ref[0, 0] = acc[0] * LOSS_SCALE


def _argmin_loss_zeros(x, emb):
    return pl.pallas_call(
        _stage1_body,
        grid=(NT, NCB),
        in_specs=[
            pl.BlockSpec((TB, EMB_DIM), lambda i, j: (i, 0)),
            pl.BlockSpec((EMB_DIM, CB), lambda i, j: (0, j)),
        ],
        out_specs=[
            pl.BlockSpec((TB,), lambda i, j: (i,)),
            pl.BlockSpec(memory_space=pltpu.SMEM),
            pl.BlockSpec(memory_space=pl.ANY),
            pl.BlockSpec((TB,), lambda i, j: (i,)),
        ],
        out_shape=[
            jax.ShapeDtypeStruct((TOKENS,), jnp.int32),
            jax.ShapeDtypeStruct((1, 1), jnp.float32),
            jax.ShapeDtypeStruct((TOKENS, CODEBOOK), jnp.float32),
            jax.ShapeDtypeStruct((TOKENS,), jnp.int32),
        ],
        scratch_shapes=[
            pltpu.VMEM((TB, 128), jnp.float32),
            pltpu.VMEM((TB, 128), jnp.int32),
            pltpu.SMEM((1,), jnp.float32),
            pltpu.VMEM((ZROWS, CODEBOOK), jnp.float32),
            pltpu.SemaphoreType.DMA,
        ],
        compiler_params=pltpu.CompilerParams(
            dimension_semantics=("arbitrary", "arbitrary")),
        name="stage1",
    )(x, emb)


def _sc_gather_scatter(table, idx, rows64, eye128, enc_rows128):
    info = plsc.get_sparse_core_info()
    nc, ns = info.num_cores, info.num_subcores
    nw = nc * ns
    bpw = TOKENS // nw
    mesh = plsc.VectorSubcoreMesh(core_axis_name="c", subcore_axis_name="s")

    @functools.partial(
        pl.kernel, mesh=mesh,
        out_type=jax.ShapeDtypeStruct((TOKENS, EMB_DIM), jnp.float32),
        scratch_types=[
            pltpu.VMEM((bpw,), jnp.int32),
            pltpu.VMEM((bpw, EMB_DIM), jnp.float32),
            pltpu.VMEM((bpw,), jnp.int32),
            pltpu.VMEM((bpw,), jnp.int32),
            pltpu.VMEM((bpw, 128), jnp.float32),
            pltpu.SemaphoreType.DMA,
            pltpu.SemaphoreType.DMA,
        ],
    )
    def gather_k(table_hbm, idx_hbm, rows64_hbm, eye_hbm, enc_hbm, out_hbm,
                 idx_v, rows_v, row64_v, cols_v, pat_v, sem, sem2):
        wid = lax.axis_index("s") * nc + lax.axis_index("c")
        base = wid * bpw
        pltpu.sync_copy(idx_hbm.at[pl.ds(base, bpw)], idx_v)
        pltpu.sync_copy(rows64_hbm.at[pl.ds(base, bpw)], row64_v)
        gat = pltpu.async_copy(table_hbm.at[idx_v], rows_v, sem)

        for q in range(bpw // 16):
            off = q * 16
            cols_v[pl.ds(off, 16)] = idx_v[pl.ds(off, 16)] & 127

        pltpu.async_copy(eye_hbm.at[cols_v], pat_v, sem2).wait()
        gat.wait()
        pltpu.sync_copy(rows_v, out_hbm.at[pl.ds(base, bpw)])
        pltpu.async_copy(pat_v, enc_hbm.at[row64_v], sem2).wait()

    return gather_k(table, idx, rows64, eye128, enc_rows128)


def kernel(inputs, embeddings):
    x = inputs.reshape(-1, EMB_DIM)
    idx, loss11, enc0, rows64 = _argmin_loss_zeros(x, embeddings)
    emb_t = jnp.swapaxes(embeddings, 0, 1)
    enc_rows128 = enc0.reshape(TOKENS * (CODEBOOK // 128), 128)
    eye128 = jnp.eye(128, dtype=jnp.float32)
    quantized = _sc_gather_scatter(emb_t, idx, rows64, eye128, enc_rows128)
    encodings, quantized = lax.optimization_barrier((enc0, quantized))
    quantized_st = quantized.reshape(inputs.shape)
    encoding_indices = idx.reshape(inputs.shape[:-1])
    loss = loss11[0, 0]
    return quantized_st, encodings, encoding_indices, loss


# speedup vs baseline: 8.1069x; 2.4552x over previous
import functools

import jax
import jax.numpy as jnp
from jax import lax
from jax.experimental import pallas as pl
from jax.experimental.pallas import tpu as pltpu
from jax.experimental.pallas import tpu_sc as plsc

EMB_DIM = 256
CODEBOOK = 8192
TOKENS = 8192
TB = 1024
CB = 1024
NT = TOKENS // TB
NCB = CODEBOOK // CB
OH_TB = 256
LOSS_SCALE = 1.25 / (TOKENS * EMB_DIM)


def _stage1_body(x_ref, e_ref, idx_ref, loss_ref, minv, mini, acc):
    i = pl.program_id(0)
    j = pl.program_id(1)

    @pl.when(jnp.logical_and(i == 0, j == 0))
    def _():
        acc[0] = jnp.float32(0.0)

    @pl.when(j == 0)
    def _():
        minv[...] = jnp.full((TB, 128), jnp.inf, jnp.float32)
        mini[...] = jnp.zeros((TB, 128), jnp.int32)

    xb = x_ref[...]
    eb = e_ref[...]
    s2 = lax.dot_general(xb, eb + eb, (((1,), (0,)), ((), ())),
                         preferred_element_type=jnp.float32)
    a = jnp.sum(xb * xb, axis=1, keepdims=True)
    b = jnp.sum(eb * eb, axis=0)
    m = minv[...]
    ii = mini[...]
    for k in range(CB // 128):
        sk = lax.slice(s2, (0, k * 128), (TB, (k + 1) * 128))
        bk = lax.slice(b, (k * 128,), ((k + 1) * 128,))
        dk = (a + bk) - sk
        cond = dk < m
        m = jnp.where(cond, dk, m)
        ii = jnp.where(cond, jnp.int32(j * (CB // 128) + k), ii)
    minv[...] = m
    mini[...] = ii

    @pl.when(j == NCB - 1)
    def _():
        lane = lax.broadcasted_iota(jnp.int32, (TB, 128), 1)
        gmin = jnp.min(m, axis=1)
        cand = jnp.where(m == gmin[:, None], ii * 128 + lane,
                         jnp.int32(0x7FFFFFFF))
        idx_ref[...] = jnp.min(cand, axis=1)
        acc[0] = acc[0] + jnp.sum(gmin)

    @pl.when(jnp.logical_and(i == NT - 1, j == NCB - 1))
    def _():
        loss_ref[0, 0] = acc[0] * LOSS_SCALE


def _argmin_loss(x, emb):
    return pl.pallas_call(
        _stage1_body,
        grid=(NT, NCB),
        in_specs=[
            pl.BlockSpec((TB, EMB_DIM), lambda i, j: (i, 0)),
            pl.BlockSpec((EMB_DIM, CB), lambda i, j: (0, j)),
        ],
        out_specs=[
            pl.BlockSpec((TB,), lambda i, j: (i,)),
            pl.BlockSpec(memory_space=pltpu.SMEM),
        ],
        out_shape=[
            jax.ShapeDtypeStruct((TOKENS,), jnp.int32),
            jax.ShapeDtypeStruct((1, 1), jnp.float32),
        ],
        scratch_shapes=[
            pltpu.VMEM((TB, 128), jnp.float32),
            pltpu.VMEM((TB, 128), jnp.int32),
            pltpu.SMEM((1,), jnp.float32),
        ],
        compiler_params=pltpu.CompilerParams(
            dimension_semantics=("arbitrary", "arbitrary")),
        name="vq_argmin",
    )(x, emb)


def _onehot_body(idx_ref, out_ref):
    ids = idx_ref[...]
    cols = lax.broadcasted_iota(jnp.int32, (OH_TB, CODEBOOK), 1)
    out_ref[...] = (ids[:, None] == cols).astype(jnp.float32)


def _onehot(idx):
    return pl.pallas_call(
        _onehot_body,
        grid=(TOKENS // OH_TB,),
        in_specs=[pl.BlockSpec((OH_TB,), lambda i: (i,))],
        out_specs=pl.BlockSpec((OH_TB, CODEBOOK), lambda i: (i, 0)),
        out_shape=jax.ShapeDtypeStruct((TOKENS, CODEBOOK), jnp.float32),
        compiler_params=pltpu.CompilerParams(
            dimension_semantics=("arbitrary",)),
        name="vq_onehot",
    )(idx)


def _sc_gather(table, idx):
    info = plsc.get_sparse_core_info()
    nc, ns = info.num_cores, info.num_subcores
    nw = nc * ns
    b_per_w = TOKENS // nw
    mesh = plsc.VectorSubcoreMesh(core_axis_name="c", subcore_axis_name="s")

    @functools.partial(
        pl.kernel, mesh=mesh,
        out_type=jax.ShapeDtypeStruct((TOKENS, EMB_DIM), jnp.float32),
        scratch_types=[
            pltpu.VMEM((b_per_w,), jnp.int32),
            pltpu.VMEM((b_per_w, EMB_DIM), jnp.float32),
            pltpu.SemaphoreType.DMA,
        ],
    )
    def gather_k(table_hbm, idx_hbm, out_hbm, idx_v, rows_v, sem):
        wid = lax.axis_index("s") * nc + lax.axis_index("c")
        base = wid * b_per_w
        pltpu.sync_copy(idx_hbm.at[pl.ds(base, b_per_w)], idx_v)
        pltpu.async_copy(table_hbm.at[idx_v], rows_v, sem).wait()
        pltpu.sync_copy(rows_v, out_hbm.at[pl.ds(base, b_per_w)])

    return gather_k(table, idx)


def kernel(inputs, embeddings):
    x = inputs.reshape(-1, EMB_DIM)
    idx, loss11 = _argmin_loss(x, embeddings)
    emb_t = jnp.swapaxes(embeddings, 0, 1)
    quantized = _sc_gather(emb_t, idx)
    encodings = _onehot(idx)
    quantized_st = quantized.reshape(inputs.shape)
    encoding_indices = idx.reshape(inputs.shape[:-1])
    loss = loss11[0, 0]
    return quantized_st, encodings, encoding_indices, loss
